# baseline (speedup 1.0000x reference)
import jax
import jax.numpy as jnp
from jax import lax
from jax.experimental import pallas as pl
from jax.experimental.pallas import tpu as pltpu
from jax.experimental.pallas import tpu_sc as plsc

NUM_USERS = 5000
NUM_ITEMS = 5000
N_NODES = NUM_USERS + NUM_ITEMS
N_EDGES = 320000
EMB = 128
LANES = 16

NC = 2
NS = 16
NW = NC * NS
E_PER_W = N_EDGES // NW
K = 80
NCHUNK = 128
E_PAD_W = NCHUNK * K
NBUF = 3
ROWS_MAIN = 624
ROWS_LAST = N_NODES - (NS - 1) * ROWS_MAIN
ZCH = 16


def _splat_lane(vec, i):
  idx = jnp.full((LANES, 1), i, jnp.int32)
  dnums = lax.GatherDimensionNumbers(
      offset_dims=(), collapsed_slice_dims=(0,), start_index_map=(0,))
  return lax.gather(vec, idx, dnums, slice_sizes=(1,),
                    mode=lax.GatherScatterMode.PROMISE_IN_BOUNDS)


def _sc_layer_body(ego_hbm, src_hbm, dst_hbm, w_hbm, out_hbm,
                   src_v, dst_v, wch_v, rows_v, zero_v, acc_sh,
                   gsem, ssem, dsem, wsem, lsem):
  cid = lax.axis_index("c")
  sid = lax.axis_index("s")
  wid = sid * NC + cid

  ld_src = pltpu.async_copy(src_hbm.at[wid], src_v, lsem)

  for r in range(ZCH):
    for j in range(EMB // LANES):
      zero_v[r, pl.ds(j * LANES, LANES)] = jnp.zeros((LANES,), jnp.float32)

  tbase = pl.multiple_of(sid * ROWS_MAIN, 8)

  def zrow(r, _):
    pltpu.sync_copy(zero_v, acc_sh.at[pl.ds(tbase + r * ZCH, ZCH)])
    return ()
  lax.fori_loop(0, ROWS_MAIN // ZCH, zrow, ())

  @pl.when(sid == NS - 1)
  def _():
    for r in range(ROWS_MAIN // ZCH, ROWS_LAST // ZCH):
      pltpu.sync_copy(zero_v, acc_sh.at[pl.ds(tbase + r * ZCH, ZCH)])

  ld_src.wait()
  plsc.subcore_barrier()

  def dw_start(c, b):
    pltpu.async_copy(dst_hbm.at[wid].at[c], dst_v.at[b], dsem.at[b])
    pltpu.async_copy(w_hbm.at[wid].at[c], wch_v.at[b], wsem.at[b])

  def dw_wait(c, b):
    pltpu.make_async_copy(dst_hbm.at[wid].at[c], dst_v.at[b],
                          dsem.at[b]).wait()
    pltpu.make_async_copy(w_hbm.at[wid].at[c], wch_v.at[b],
                          wsem.at[b]).wait()

  def gather_start(c, b):
    pltpu.async_copy(ego_hbm.at[src_v.at[pl.ds(c * K, K)]],
                     rows_v.at[b], gsem.at[b])

  def gather_wait(c, b):
    pltpu.make_async_copy(ego_hbm.at[src_v.at[pl.ds(c * K, K)]],
                          rows_v.at[b], gsem.at[b]).wait()

  def scatter_start(c, b):
    pltpu.async_copy(rows_v.at[b], acc_sh.at[dst_v.at[b].at[0]],
                     ssem.at[b], add=True)

  def scatter_wait(c, b):
    pltpu.make_async_copy(rows_v.at[b], acc_sh.at[dst_v.at[b].at[0]],
                          ssem.at[b]).wait()

  def multiply(b):
    def group(g, _):
      wv = wch_v[b, 0, pl.ds(g * LANES, LANES)]
      for i in range(LANES):
        e = g * LANES + i
        wsp = _splat_lane(wv, i)
        for j in range(EMB // LANES):
          rows_v[b, e, pl.ds(j * LANES, LANES)] = (
              rows_v[b, e, pl.ds(j * LANES, LANES)] * wsp)
      return ()
    lax.fori_loop(0, K // LANES, group, ())

  def step(c, b, do_next):
    dw_wait(c, b)
    gather_wait(c, b)
    multiply(b)
    scatter_start(c, b)
    bprev = (b + NBUF - 1) % NBUF
    scatter_wait(c - 1, bprev)
    if do_next:
      dw_start(c + 2, bprev)
      gather_start(c + 2, bprev)

  dw_start(0, 0)
  dw_start(1, 1)
  gather_start(0, 0)
  gather_start(1, 1)

  dw_wait(0, 0)
  gather_wait(0, 0)
  multiply(0)
  scatter_start(0, 0)
  dw_start(2, 2)
  gather_start(2, 2)

  NSTEADY = (NCHUNK - 5) // NBUF

  def ring(t, _):
    c0 = 1 + t * NBUF
    for bb in range(NBUF):
      step(c0 + bb, (1 + bb) % NBUF, True)
    return ()
  lax.fori_loop(0, NSTEADY, ring, ())
  for c in range(1 + NSTEADY * NBUF, NCHUNK):
    step(c, c % NBUF, c + 2 < NCHUNK)
  scatter_wait(NCHUNK - 1, (NCHUNK - 1) % NBUF)
  plsc.subcore_barrier()

  @pl.when(sid < NS - 1)
  def _():
    pltpu.sync_copy(acc_sh.at[pl.ds(tbase, ROWS_MAIN)],
                    out_hbm.at[cid].at[pl.ds(tbase, ROWS_MAIN)])

  @pl.when(sid == NS - 1)
  def _():
    pltpu.sync_copy(acc_sh.at[pl.ds(tbase, ROWS_LAST)],
                    out_hbm.at[cid].at[pl.ds(tbase, ROWS_LAST)])


def _sc_layer(ego, src2, dst4, w4):
  mesh = plsc.VectorSubcoreMesh(core_axis_name="c", subcore_axis_name="s")
  f = pl.kernel(
      _sc_layer_body,
      out_type=jax.ShapeDtypeStruct((NC, N_NODES, EMB), jnp.float32),
      mesh=mesh,
      scratch_types=[
          pltpu.VMEM((E_PAD_W,), jnp.int32),
          pltpu.VMEM((NBUF, 1, K), jnp.int32),
          pltpu.VMEM((NBUF, 1, K), jnp.float32),
          pltpu.VMEM((NBUF, K, EMB), jnp.float32),
          pltpu.VMEM((ZCH, EMB), jnp.float32),
          pltpu.VMEM_SHARED((N_NODES, EMB), jnp.float32),
          pltpu.SemaphoreType.DMA((NBUF,)),
          pltpu.SemaphoreType.DMA((NBUF,)),
          pltpu.SemaphoreType.DMA((NBUF,)),
          pltpu.SemaphoreType.DMA((NBUF,)),
          pltpu.SemaphoreType.DMA,
      ],
  )
  return f(ego, src2, dst4, w4)


def _tc_add2(a, b):
  def body(a_ref, b_ref, o_ref):
    o_ref[...] = a_ref[...] + b_ref[...]
  grid = 10
  blk = N_NODES // grid
  spec = pl.BlockSpec((blk, EMB), lambda i: (i, 0))
  return pl.pallas_call(
      body,
      out_shape=jax.ShapeDtypeStruct((N_NODES, EMB), jnp.float32),
      grid=(grid,),
      in_specs=[spec, spec],
      out_specs=spec,
  )(a, b)


def _tc_mean3(e0, e1, p0, p1):
  def body(a_ref, b_ref, c_ref, d_ref, o_ref):
    o_ref[...] = (a_ref[...] + b_ref[...] + c_ref[...] + d_ref[...]) * (
        jnp.float32(1.0 / 3.0))
  grid = 10
  blk = N_NODES // grid
  spec = pl.BlockSpec((blk, EMB), lambda i: (i, 0))
  return pl.pallas_call(
      body,
      out_shape=jax.ShapeDtypeStruct((N_NODES, EMB), jnp.float32),
      grid=(grid,),
      in_specs=[spec, spec, spec, spec],
      out_specs=spec,
  )(e0, e1, p0, p1)


def kernel(user_emb, item_emb, edge_weight, edge_index):
  ego0 = jnp.concatenate([user_emb, item_emb], axis=0)
  npad = E_PAD_W - E_PER_W
  zpad_i = jnp.zeros((NW, npad), jnp.int32)
  padvals = ((jnp.arange(npad, dtype=jnp.int32)[None, :] * 37
              + jnp.arange(NW, dtype=jnp.int32)[:, None] * 389) % N_NODES)
  src2 = jnp.concatenate(
      [edge_index[0].astype(jnp.int32).reshape(NW, E_PER_W), zpad_i], axis=1)
  dst4 = jnp.concatenate(
      [edge_index[1].astype(jnp.int32).reshape(NW, E_PER_W), padvals],
      axis=1).reshape(NW, NCHUNK, 1, K)
  w4 = jnp.concatenate(
      [edge_weight.astype(jnp.float32).reshape(NW, E_PER_W),
       jnp.zeros((NW, npad), jnp.float32)], axis=1).reshape(NW, NCHUNK, 1, K)

  p = _sc_layer(ego0, src2, dst4, w4)
  ego1 = _tc_add2(p[0], p[1])
  q = _sc_layer(ego1, src2, dst4, w4)
  mean_emb = _tc_mean3(ego0, ego1, q[0], q[1])
  return (mean_emb[:NUM_USERS], mean_emb[NUM_USERS:])


# speedup vs baseline: 2.7525x; 2.7525x over previous
import jax
import jax.numpy as jnp
from jax import lax
from jax.experimental import pallas as pl
from jax.experimental.pallas import tpu as pltpu
from jax.experimental.pallas import tpu_sc as plsc

NUM_USERS = 5000
NUM_ITEMS = 5000
N_NODES = NUM_USERS + NUM_ITEMS
N_EDGES = 320000
EMB = 128
LANES = 16

NC = 2
NS = 16
NW = NC * NS
E_PER_W = N_EDGES // NW
K = 80
NCHUNK = 128
E_PAD_W = NCHUNK * K
NBUF = 3
ROWS_MAIN = 624
ROWS_LAST = N_NODES - (NS - 1) * ROWS_MAIN
ZCH = 16


def _splat_lane(vec, i):
  idx = jnp.full((LANES, 1), i, jnp.int32)
  dnums = lax.GatherDimensionNumbers(
      offset_dims=(), collapsed_slice_dims=(0,), start_index_map=(0,))
  return lax.gather(vec, idx, dnums, slice_sizes=(1,),
                    mode=lax.GatherScatterMode.PROMISE_IN_BOUNDS)


def _sc_layer_body(ego_hbm, src_hbm, dst_hbm, w_hbm, out_hbm,
                   src_v, dst_v, wch_v, rows_v, zero_v, acc_sh,
                   gsem, ssem, dsem, wsem, lsem):
  cid = lax.axis_index("c")
  sid = lax.axis_index("s")
  wid = sid * NC + cid

  ld_src = pltpu.async_copy(src_hbm.at[wid], src_v, lsem)

  for r in range(ZCH):
    for j in range(EMB // LANES):
      zero_v[r, pl.ds(j * LANES, LANES)] = jnp.zeros((LANES,), jnp.float32)

  tbase = pl.multiple_of(sid * ROWS_MAIN, 8)

  def zrow(r, _):
    pltpu.sync_copy(zero_v, acc_sh.at[pl.ds(tbase + r * ZCH, ZCH)])
    return ()
  lax.fori_loop(0, ROWS_MAIN // ZCH, zrow, ())

  @pl.when(sid == NS - 1)
  def _():
    for r in range(ROWS_MAIN // ZCH, ROWS_LAST // ZCH):
      pltpu.sync_copy(zero_v, acc_sh.at[pl.ds(tbase + r * ZCH, ZCH)])

  ld_src.wait()
  plsc.subcore_barrier()

  def dw_start(c, b):
    pltpu.async_copy(dst_hbm.at[wid].at[c], dst_v.at[b], dsem.at[b])
    pltpu.async_copy(w_hbm.at[wid].at[c], wch_v.at[b], wsem.at[b])

  def dw_wait(c, b):
    pltpu.make_async_copy(dst_hbm.at[wid].at[c], dst_v.at[b],
                          dsem.at[b]).wait()
    pltpu.make_async_copy(w_hbm.at[wid].at[c], wch_v.at[b],
                          wsem.at[b]).wait()

  def gather_start(c, b):
    pltpu.async_copy(ego_hbm.at[src_v.at[pl.ds(c * K, K)]],
                     rows_v.at[b], gsem.at[b])

  def gather_wait(c, b):
    pltpu.make_async_copy(ego_hbm.at[src_v.at[pl.ds(c * K, K)]],
                          rows_v.at[b], gsem.at[b]).wait()

  def scatter_start(c, b):
    pltpu.async_copy(rows_v.at[b], acc_sh.at[dst_v.at[b].at[0]],
                     ssem.at[b], add=True)

  def scatter_wait(c, b):
    pltpu.make_async_copy(rows_v.at[b], acc_sh.at[dst_v.at[b].at[0]],
                          ssem.at[b]).wait()

  def multiply(b):
    def group(g, _):
      wv = wch_v[b, 0, pl.ds(g * LANES, LANES)]
      for i in range(LANES):
        e = g * LANES + i
        wsp = _splat_lane(wv, i)
        for j in range(EMB // LANES):
          rows_v[b, e, pl.ds(j * LANES, LANES)] = (
              rows_v[b, e, pl.ds(j * LANES, LANES)] * wsp)
      return ()
    lax.fori_loop(0, K // LANES, group, ())

  def step(c, b, do_next):
    dw_wait(c, b)
    gather_wait(c, b)
    multiply(b)
    scatter_start(c, b)
    bprev = (b + NBUF - 1) % NBUF
    scatter_wait(c - 1, bprev)
    if do_next:
      dw_start(c + 2, bprev)
      gather_start(c + 2, bprev)

  dw_start(0, 0)
  dw_start(1, 1)
  gather_start(0, 0)
  gather_start(1, 1)

  dw_wait(0, 0)
  gather_wait(0, 0)
  multiply(0)
  scatter_start(0, 0)
  dw_start(2, 2)
  gather_start(2, 2)

  NSTEADY = (NCHUNK - 5) // NBUF

  def ring(t, _):
    c0 = 1 + t * NBUF
    for bb in range(NBUF):
      step(c0 + bb, (1 + bb) % NBUF, True)
    return ()
  lax.fori_loop(0, NSTEADY, ring, ())
  for c in range(1 + NSTEADY * NBUF, NCHUNK):
    step(c, c % NBUF, c + 2 < NCHUNK)
  scatter_wait(NCHUNK - 1, (NCHUNK - 1) % NBUF)
  plsc.subcore_barrier()

  @pl.when(sid < NS - 1)
  def _():
    pltpu.sync_copy(acc_sh.at[pl.ds(tbase, ROWS_MAIN)],
                    out_hbm.at[cid].at[pl.ds(tbase, ROWS_MAIN)])

  @pl.when(sid == NS - 1)
  def _():
    pltpu.sync_copy(acc_sh.at[pl.ds(tbase, ROWS_LAST)],
                    out_hbm.at[cid].at[pl.ds(tbase, ROWS_LAST)])


def _sc_layer(ego, src2, dst4, w4):
  mesh = plsc.VectorSubcoreMesh(core_axis_name="c", subcore_axis_name="s")
  f = pl.kernel(
      _sc_layer_body,
      out_type=jax.ShapeDtypeStruct((NC, N_NODES, EMB), jnp.float32),
      mesh=mesh,
      scratch_types=[
          pltpu.VMEM((E_PAD_W,), jnp.int32),
          pltpu.VMEM((NBUF, 1, K), jnp.int32),
          pltpu.VMEM((NBUF, 1, K), jnp.float32),
          pltpu.VMEM((NBUF, K, EMB), jnp.float32),
          pltpu.VMEM((ZCH, EMB), jnp.float32),
          pltpu.VMEM_SHARED((N_NODES, EMB), jnp.float32),
          pltpu.SemaphoreType.DMA((NBUF,)),
          pltpu.SemaphoreType.DMA((NBUF,)),
          pltpu.SemaphoreType.DMA((NBUF,)),
          pltpu.SemaphoreType.DMA((NBUF,)),
          pltpu.SemaphoreType.DMA,
      ],
  )
  return f(ego, src2, dst4, w4)


def _tc_add2(a, b):
  def body(a_ref, b_ref, o_ref):
    o_ref[...] = a_ref[...] + b_ref[...]
  grid = 10
  blk = N_NODES // grid
  spec = pl.BlockSpec((blk, EMB), lambda i: (i, 0))
  return pl.pallas_call(
      body,
      out_shape=jax.ShapeDtypeStruct((N_NODES, EMB), jnp.float32),
      grid=(grid,),
      in_specs=[spec, spec],
      out_specs=spec,
  )(a, b)


def _tc_mean3(e0, e1, p0, p1):
  def body(a_ref, b_ref, c_ref, d_ref, o_ref):
    o_ref[...] = (a_ref[...] + b_ref[...] + c_ref[...] + d_ref[...]) * (
        jnp.float32(1.0 / 3.0))
  grid = 10
  blk = N_NODES // grid
  spec = pl.BlockSpec((blk, EMB), lambda i: (i, 0))
  return pl.pallas_call(
      body,
      out_shape=jax.ShapeDtypeStruct((N_NODES, EMB), jnp.float32),
      grid=(grid,),
      in_specs=[spec, spec, spec, spec],
      out_specs=spec,
  )(e0, e1, p0, p1)


def kernel(user_emb, item_emb, edge_weight, edge_index):
  ego0 = jnp.concatenate([user_emb, item_emb], axis=0)
  npad = E_PAD_W - E_PER_W
  padvals = ((jnp.arange(npad, dtype=jnp.int32)[None, :] * 37
              + jnp.arange(NW, dtype=jnp.int32)[:, None] * 389) % N_NODES)
  src2 = jnp.concatenate(
      [edge_index[0].astype(jnp.int32).reshape(NW, E_PER_W), padvals], axis=1)
  dst4 = jnp.concatenate(
      [edge_index[1].astype(jnp.int32).reshape(NW, E_PER_W), padvals],
      axis=1).reshape(NW, NCHUNK, 1, K)
  w4 = jnp.concatenate(
      [edge_weight.astype(jnp.float32).reshape(NW, E_PER_W),
       jnp.zeros((NW, npad), jnp.float32)], axis=1).reshape(NW, NCHUNK, 1, K)

  p = _sc_layer(ego0, src2, dst4, w4)
  ego1 = _tc_add2(p[0], p[1])
  q = _sc_layer(ego1, src2, dst4, w4)
  mean_emb = _tc_mean3(ego0, ego1, q[0], q[1])
  return (mean_emb[:NUM_USERS], mean_emb[NUM_USERS:])
